# baseline (speedup 1.0000x reference)
import functools

import jax
import jax.numpy as jnp
from jax import lax
from jax.experimental import pallas as pl
from jax.experimental.pallas import tpu as pltpu
from jax.experimental.pallas import tpu_sc as plsc

_LANES = 128
_ROWS = 1800
_CHUNK = 6480


def _sc_body(u_hbm, v_hbm, calib_hbm, words_hbm,
             u_v, v_v, calib_v, packed_v, words_v):
    n_workers = 32
    n_pix = u_hbm.shape[0]
    share = n_pix // n_workers
    wid = lax.axis_index("s") * 2 + lax.axis_index("c")
    base = wid * share

    pltpu.sync_copy(calib_hbm, calib_v)
    lane = lax.iota(jnp.int32, 16)

    def pack_step(eb, carry):
        e16 = (eb * 16 + lane) * 16
        acc = jnp.zeros((16,), jnp.int32)
        for d in range(16):
            acc = acc | (plsc.load_gather(calib_v, [e16 + d]) << d)
        packed_v[pl.ds(eb * 16, 16)] = acc
        return carry

    lax.fori_loop(0, 64, pack_step, 0)

    def chunk_step(ci, carry):
        off = base + ci * _CHUNK
        pltpu.sync_copy(u_hbm.at[pl.ds(off, _CHUNK)], u_v)
        pltpu.sync_copy(v_hbm.at[pl.ds(off, _CHUNK)], v_v)

        def vec_step(k, c2):
            ub = u_v[pl.ds(k * 16, 16)]
            vb = v_v[pl.ds(k * 16, 16)]
            idx = (ub << 5) + vb
            words_v[pl.ds(k * 16, 16)] = plsc.load_gather(packed_v, [idx])
            return c2

        lax.fori_loop(0, _CHUNK // 16, vec_step, 0, unroll=8)
        pltpu.sync_copy(words_v, words_hbm.at[pl.ds(off, _CHUNK)])
        return carry

    lax.fori_loop(0, share // _CHUNK, chunk_step, 0)


def _sc_gather_words(u_flat, v_flat, calib_flat):
    n_pix = u_flat.shape[0]
    mesh = plsc.VectorSubcoreMesh(core_axis_name="c", subcore_axis_name="s")
    f = functools.partial(
        pl.kernel, _sc_body, mesh=mesh,
        out_type=jax.ShapeDtypeStruct((n_pix,), jnp.int32),
        compiler_params=pltpu.CompilerParams(needs_layout_passes=False),
        scratch_types=[
            pltpu.VMEM((_CHUNK,), jnp.int32),
            pltpu.VMEM((_CHUNK,), jnp.int32),
            pltpu.VMEM((calib_flat.shape[0],), jnp.int32),
            pltpu.VMEM((1024,), jnp.int32),
            pltpu.VMEM((_CHUNK,), jnp.int32),
        ],
    )()
    return f(u_flat, v_flat, calib_flat)


def _f32(x):
    return jax.lax.bitcast_convert_type(x, jnp.float32)


def _tc_body(d_ref, uv_ref, words_ref, coef_ref, out_ref):
    nb = d_ref.shape[0]
    shape = uv_ref.shape

    words = words_ref[...]
    uv = uv_ref[...]
    c01 = jnp.broadcast_to(coef_ref[0:1, :], shape)
    c23 = jnp.broadcast_to(coef_ref[1:2, :], shape)
    himask = jnp.int32(-65536)

    for b in range(nb):
        t = d_ref[b] * 16.0
        tf = jnp.floor(t)
        i = tf.astype(jnp.int32)
        u = t - tf
        g01 = jnp.take_along_axis(c01, i, axis=1)
        g23 = jnp.take_along_axis(c23, i, axis=1)
        a0 = _f32(g01 << 16)
        a1 = _f32(g01 & himask)
        a2 = _f32(g23 << 16)
        a3 = _f32(g23 & himask)
        d_comp = a0 + u * (a1 + u * (a2 + u * a3))
        ok = ((words >> i) & 1) == 1
        out_ref[b] = jnp.where(ok, d_comp * uv, 0.0)


@jax.jit
def kernel(d_map, uv_comp, u_cell_ids, v_cell_ids, cell_is_calib, d_ctrl):
    B, H, W = d_map.shape
    UN, VN, DN = cell_is_calib.shape
    n_pix = H * W
    rows = n_pix // _LANES

    words_flat = _sc_gather_words(
        u_cell_ids.reshape(n_pix),
        v_cell_ids.reshape(n_pix),
        cell_is_calib.astype(jnp.int32).reshape(UN * VN * DN),
    )

    d2 = d_map.reshape(B, rows, _LANES)
    uv2 = uv_comp.reshape(rows, _LANES)
    words2 = words_flat.reshape(rows, _LANES)

    p0, p1 = d_ctrl[0:DN], d_ctrl[1:DN + 1]
    p2, p3 = d_ctrl[2:DN + 2], d_ctrl[3:DN + 3]
    a0 = (p0 + 4.0 * p1 + p2) / 6.0
    a1 = (p2 - p0) / 2.0
    a2 = (p0 - 2.0 * p1 + p2) / 2.0
    a3 = (p3 - p0) / 6.0 + (p1 - p2) / 2.0

    def _pair(lo, hi_):
        lo16 = jax.lax.bitcast_convert_type(
            lo.astype(jnp.bfloat16), jnp.uint16).astype(jnp.int32)
        hi16 = jax.lax.bitcast_convert_type(
            hi_.astype(jnp.bfloat16), jnp.uint16).astype(jnp.int32)
        return lo16 | (hi16 << 16)

    coef = jnp.zeros((2, _LANES), jnp.int32)
    coef = coef.at[0, :DN].set(_pair(a0, a1))
    coef = coef.at[1, :DN].set(_pair(a2, a3))

    grid = (rows // _ROWS,)
    out = pl.pallas_call(
        _tc_body,
        grid=grid,
        in_specs=[
            pl.BlockSpec((B, _ROWS, _LANES), lambda i: (0, i, 0)),
            pl.BlockSpec((_ROWS, _LANES), lambda i: (i, 0)),
            pl.BlockSpec((_ROWS, _LANES), lambda i: (i, 0)),
            pl.BlockSpec((2, _LANES), lambda i: (0, 0)),
        ],
        out_specs=pl.BlockSpec((B, _ROWS, _LANES), lambda i: (0, i, 0)),
        out_shape=jax.ShapeDtypeStruct((B, rows, _LANES), jnp.float32),
        compiler_params=pltpu.CompilerParams(
            dimension_semantics=("arbitrary",),
        ),
    )(d2, uv2, words2, coef)
    return out.reshape(B, H, W)


# speedup vs baseline: 1.0033x; 1.0033x over previous
import functools

import jax
import jax.numpy as jnp
from jax import lax
from jax.experimental import pallas as pl
from jax.experimental.pallas import tpu as pltpu
from jax.experimental.pallas import tpu_sc as plsc

_LANES = 128
_ROWS = 648
_CHUNK = 6480


def _sc_body(u_hbm, v_hbm, calib_hbm, words_hbm,
             u_v, v_v, calib_v, packed_v, words_v):
    n_workers = 32
    n_pix = u_hbm.shape[0]
    share = n_pix // n_workers
    wid = lax.axis_index("s") * 2 + lax.axis_index("c")
    base = wid * share

    pltpu.sync_copy(calib_hbm, calib_v)
    lane = lax.iota(jnp.int32, 16)

    def pack_step(eb, carry):
        e16 = (eb * 16 + lane) * 16
        acc = jnp.zeros((16,), jnp.int32)
        for d in range(16):
            acc = acc | (plsc.load_gather(calib_v, [e16 + d]) << d)
        packed_v[pl.ds(eb * 16, 16)] = acc
        return carry

    lax.fori_loop(0, 64, pack_step, 0)

    def chunk_step(ci, carry):
        off = base + ci * _CHUNK
        pltpu.sync_copy(u_hbm.at[pl.ds(off, _CHUNK)], u_v)
        pltpu.sync_copy(v_hbm.at[pl.ds(off, _CHUNK)], v_v)

        def vec_step(k, c2):
            ub = u_v[pl.ds(k * 16, 16)]
            vb = v_v[pl.ds(k * 16, 16)]
            idx = (ub << 5) + vb
            words_v[pl.ds(k * 16, 16)] = plsc.load_gather(packed_v, [idx])
            return c2

        lax.fori_loop(0, _CHUNK // 16, vec_step, 0, unroll=8)
        pltpu.sync_copy(words_v, words_hbm.at[pl.ds(off, _CHUNK)])
        return carry

    lax.fori_loop(0, share // _CHUNK, chunk_step, 0)


def _sc_gather_words(u_flat, v_flat, calib_flat):
    n_pix = u_flat.shape[0]
    mesh = plsc.VectorSubcoreMesh(core_axis_name="c", subcore_axis_name="s")
    f = functools.partial(
        pl.kernel, _sc_body, mesh=mesh,
        out_type=jax.ShapeDtypeStruct((n_pix,), jnp.int32),
        compiler_params=pltpu.CompilerParams(needs_layout_passes=False),
        scratch_types=[
            pltpu.VMEM((_CHUNK,), jnp.int32),
            pltpu.VMEM((_CHUNK,), jnp.int32),
            pltpu.VMEM((calib_flat.shape[0],), jnp.int32),
            pltpu.VMEM((1024,), jnp.int32),
            pltpu.VMEM((_CHUNK,), jnp.int32),
        ],
    )()
    return f(u_flat, v_flat, calib_flat)


def _f32(x):
    return jax.lax.bitcast_convert_type(x, jnp.float32)


def _tc_body(d_ref, uv_ref, words_ref, coef_ref, out_ref):
    nb = d_ref.shape[0]
    shape = uv_ref.shape

    words = words_ref[...]
    uv = uv_ref[...]
    c01 = jnp.broadcast_to(coef_ref[0:1, :], shape)
    c23 = jnp.broadcast_to(coef_ref[1:2, :], shape)
    himask = jnp.int32(-65536)

    for b in range(nb):
        t = d_ref[b] * 16.0
        tf = jnp.floor(t)
        i = tf.astype(jnp.int32)
        u = t - tf
        g01 = jnp.take_along_axis(c01, i, axis=1)
        g23 = jnp.take_along_axis(c23, i, axis=1)
        a0 = _f32(g01 << 16)
        a1 = _f32(g01 & himask)
        a2 = _f32(g23 << 16)
        a3 = _f32(g23 & himask)
        d_comp = a0 + u * (a1 + u * (a2 + u * a3))
        ok = ((words >> i) & 1) == 1
        out_ref[b] = jnp.where(ok, d_comp * uv, 0.0)


@jax.jit
def kernel(d_map, uv_comp, u_cell_ids, v_cell_ids, cell_is_calib, d_ctrl):
    B, H, W = d_map.shape
    UN, VN, DN = cell_is_calib.shape
    n_pix = H * W
    rows = n_pix // _LANES

    words_flat = _sc_gather_words(
        u_cell_ids.reshape(n_pix),
        v_cell_ids.reshape(n_pix),
        cell_is_calib.astype(jnp.int32).reshape(UN * VN * DN),
    )

    d2 = d_map.reshape(B, rows, _LANES)
    uv2 = uv_comp.reshape(rows, _LANES)
    words2 = words_flat.reshape(rows, _LANES)

    p0, p1 = d_ctrl[0:DN], d_ctrl[1:DN + 1]
    p2, p3 = d_ctrl[2:DN + 2], d_ctrl[3:DN + 3]
    a0 = (p0 + 4.0 * p1 + p2) / 6.0
    a1 = (p2 - p0) / 2.0
    a2 = (p0 - 2.0 * p1 + p2) / 2.0
    a3 = (p3 - p0) / 6.0 + (p1 - p2) / 2.0

    def _pair(lo, hi_):
        lo16 = jax.lax.bitcast_convert_type(
            lo.astype(jnp.bfloat16), jnp.uint16).astype(jnp.int32)
        hi16 = jax.lax.bitcast_convert_type(
            hi_.astype(jnp.bfloat16), jnp.uint16).astype(jnp.int32)
        return lo16 | (hi16 << 16)

    coef = jnp.zeros((2, _LANES), jnp.int32)
    coef = coef.at[0, :DN].set(_pair(a0, a1))
    coef = coef.at[1, :DN].set(_pair(a2, a3))

    grid = (rows // _ROWS,)
    out = pl.pallas_call(
        _tc_body,
        grid=grid,
        in_specs=[
            pl.BlockSpec((B, _ROWS, _LANES), lambda i: (0, i, 0)),
            pl.BlockSpec((_ROWS, _LANES), lambda i: (i, 0)),
            pl.BlockSpec((_ROWS, _LANES), lambda i: (i, 0)),
            pl.BlockSpec((2, _LANES), lambda i: (0, 0)),
        ],
        out_specs=pl.BlockSpec((B, _ROWS, _LANES), lambda i: (0, i, 0)),
        out_shape=jax.ShapeDtypeStruct((B, rows, _LANES), jnp.float32),
        compiler_params=pltpu.CompilerParams(
            dimension_semantics=("arbitrary",),
        ),
    )(d2, uv2, words2, coef)
    return out.reshape(B, H, W)


# speedup vs baseline: 1.0230x; 1.0196x over previous
import functools

import jax
import jax.numpy as jnp
from jax import lax
from jax.experimental import pallas as pl
from jax.experimental.pallas import tpu as pltpu
from jax.experimental.pallas import tpu_sc as plsc

_LANES = 128
_ROWS = 648
_CHUNK = 16200


def _sc_body(u_hbm, v_hbm, calib_hbm, words_hbm,
             u_v, v_v, calib_v, packed_v, words_v):
    n_workers = 32
    n_pix = u_hbm.shape[0]
    share = n_pix // n_workers
    wid = lax.axis_index("s") * 2 + lax.axis_index("c")
    base = wid * share

    pltpu.sync_copy(calib_hbm, calib_v)
    lane = lax.iota(jnp.int32, 16)

    def pack_step(eb, carry):
        e16 = (eb * 16 + lane) * 16
        acc = jnp.zeros((16,), jnp.int32)
        for d in range(16):
            acc = acc | (plsc.load_gather(calib_v, [e16 + d]) << d)
        packed_v[pl.ds(eb * 16, 16)] = acc
        return carry

    lax.fori_loop(0, 64, pack_step, 0)

    def chunk_step(ci, carry):
        off = base + ci * _CHUNK
        pltpu.sync_copy(u_hbm.at[pl.ds(off, _CHUNK)], u_v)
        pltpu.sync_copy(v_hbm.at[pl.ds(off, _CHUNK)], v_v)

        def vec_step(k, c2):
            ub = u_v[pl.ds(k * 16, 16)]
            vb = v_v[pl.ds(k * 16, 16)]
            idx = (ub << 5) + vb
            words_v[pl.ds(k * 16, 16)] = plsc.load_gather(packed_v, [idx])
            return c2

        lax.fori_loop(0, _CHUNK // 16, vec_step, 0, unroll=8)
        pltpu.sync_copy(words_v, words_hbm.at[pl.ds(off, _CHUNK)])
        return carry

    lax.fori_loop(0, share // _CHUNK, chunk_step, 0)


def _sc_gather_words(u_flat, v_flat, calib_flat):
    n_pix = u_flat.shape[0]
    mesh = plsc.VectorSubcoreMesh(core_axis_name="c", subcore_axis_name="s")
    f = functools.partial(
        pl.kernel, _sc_body, mesh=mesh,
        out_type=jax.ShapeDtypeStruct((n_pix,), jnp.int32),
        compiler_params=pltpu.CompilerParams(needs_layout_passes=False),
        scratch_types=[
            pltpu.VMEM((_CHUNK,), jnp.int32),
            pltpu.VMEM((_CHUNK,), jnp.int32),
            pltpu.VMEM((calib_flat.shape[0],), jnp.int32),
            pltpu.VMEM((1024,), jnp.int32),
            pltpu.VMEM((_CHUNK,), jnp.int32),
        ],
    )()
    return f(u_flat, v_flat, calib_flat)


def _f32(x):
    return jax.lax.bitcast_convert_type(x, jnp.float32)


def _tc_body(d_ref, uv_ref, words_ref, coef_ref, out_ref):
    nb = d_ref.shape[0]
    shape = uv_ref.shape

    words = words_ref[...]
    uv = uv_ref[...]
    c01 = jnp.broadcast_to(coef_ref[0:1, :], shape)
    c23 = jnp.broadcast_to(coef_ref[1:2, :], shape)
    himask = jnp.int32(-65536)

    for b in range(nb):
        t = d_ref[b] * 16.0
        tf = jnp.floor(t)
        i = tf.astype(jnp.int32)
        u = t - tf
        g01 = jnp.take_along_axis(c01, i, axis=1)
        g23 = jnp.take_along_axis(c23, i, axis=1)
        a0 = _f32(g01 << 16)
        a1 = _f32(g01 & himask)
        a2 = _f32(g23 << 16)
        a3 = _f32(g23 & himask)
        d_comp = a0 + u * (a1 + u * (a2 + u * a3))
        ok = ((words >> i) & 1) == 1
        out_ref[b] = jnp.where(ok, d_comp * uv, 0.0)


@jax.jit
def kernel(d_map, uv_comp, u_cell_ids, v_cell_ids, cell_is_calib, d_ctrl):
    B, H, W = d_map.shape
    UN, VN, DN = cell_is_calib.shape
    n_pix = H * W
    rows = n_pix // _LANES

    words_flat = _sc_gather_words(
        u_cell_ids.reshape(n_pix),
        v_cell_ids.reshape(n_pix),
        cell_is_calib.astype(jnp.int32).reshape(UN * VN * DN),
    )

    d2 = d_map.reshape(B, rows, _LANES)
    uv2 = uv_comp.reshape(rows, _LANES)
    words2 = words_flat.reshape(rows, _LANES)

    p0, p1 = d_ctrl[0:DN], d_ctrl[1:DN + 1]
    p2, p3 = d_ctrl[2:DN + 2], d_ctrl[3:DN + 3]
    a0 = (p0 + 4.0 * p1 + p2) / 6.0
    a1 = (p2 - p0) / 2.0
    a2 = (p0 - 2.0 * p1 + p2) / 2.0
    a3 = (p3 - p0) / 6.0 + (p1 - p2) / 2.0

    def _pair(lo, hi_):
        lo16 = jax.lax.bitcast_convert_type(
            lo.astype(jnp.bfloat16), jnp.uint16).astype(jnp.int32)
        hi16 = jax.lax.bitcast_convert_type(
            hi_.astype(jnp.bfloat16), jnp.uint16).astype(jnp.int32)
        return lo16 | (hi16 << 16)

    coef = jnp.zeros((2, _LANES), jnp.int32)
    coef = coef.at[0, :DN].set(_pair(a0, a1))
    coef = coef.at[1, :DN].set(_pair(a2, a3))

    grid = (rows // _ROWS,)
    out = pl.pallas_call(
        _tc_body,
        grid=grid,
        in_specs=[
            pl.BlockSpec((B, _ROWS, _LANES), lambda i: (0, i, 0)),
            pl.BlockSpec((_ROWS, _LANES), lambda i: (i, 0)),
            pl.BlockSpec((_ROWS, _LANES), lambda i: (i, 0)),
            pl.BlockSpec((2, _LANES), lambda i: (0, 0)),
        ],
        out_specs=pl.BlockSpec((B, _ROWS, _LANES), lambda i: (0, i, 0)),
        out_shape=jax.ShapeDtypeStruct((B, rows, _LANES), jnp.float32),
        compiler_params=pltpu.CompilerParams(
            dimension_semantics=("arbitrary",),
        ),
    )(d2, uv2, words2, coef)
    return out.reshape(B, H, W)
